# baseline (speedup 1.0000x reference)
import functools

import jax
import jax.numpy as jnp
from jax import lax
from jax.experimental import pallas as pl
from jax.experimental.pallas import tpu as pltpu
from jax.experimental.pallas import tpu_sc as plsc

_BATCH = 16384
_DIM = 128
_NC = 2
_NS = 16
_NW = _NC * _NS
_BPW = _BATCH // _NW
_CHUNK = 128
_NCHUNK = _BPW // _CHUNK
_NT = 2 * _NCHUNK
_NBUF = 3
_ROWS = 513
_OFF_T = 520


def _tec_body(starts_hbm, totals_hbm, start_hbm, total_hbm,
              out_s_hbm, out_t_hbm, idx_s_v, idx_t_v, rows_v, tabs_sh,
              sem_i, sem_g, sem_o):
    cid = lax.axis_index("c")
    sid = lax.axis_index("s")
    wid = sid * _NC + cid
    base = wid * _BPW
    idx_cps = [
        pltpu.async_copy(starts_hbm.at[pl.ds(base, _BPW)], idx_s_v, sem_i),
        pltpu.async_copy(totals_hbm.at[pl.ds(base, _BPW)], idx_t_v, sem_i),
    ]

    def _idx(k):
        ref = idx_s_v if k < _NCHUNK else idx_t_v
        return ref.at[pl.ds((k % _NCHUNK) * _CHUNK, _CHUNK)]

    @pl.when(sid == 0)
    def _():
        pltpu.sync_copy(start_hbm, tabs_sh.at[pl.ds(0, _ROWS)])

    @pl.when(sid == 1)
    def _():
        pltpu.sync_copy(total_hbm, tabs_sh.at[pl.ds(_OFF_T, _ROWS)])

    for cp in idx_cps:
        cp.wait()
    plsc.subcore_barrier()

    gathers = [None] * _NT
    outs = [None] * _NT
    for k in range(_NBUF):
        gathers[k] = pltpu.async_copy(
            tabs_sh.at[_idx(k)], rows_v.at[k], sem_g.at[k])
    for k in range(_NT):
        buf = k % _NBUF
        if k >= _NBUF:
            outs[k - _NBUF].wait()
            gathers[k] = pltpu.async_copy(
                tabs_sh.at[_idx(k)], rows_v.at[buf], sem_g.at[k])
        out = out_s_hbm if k < _NCHUNK else out_t_hbm
        gathers[k].wait()
        c = k % _NCHUNK
        outs[k] = pltpu.async_copy(
            rows_v.at[buf], out.at[pl.ds(base + c * _CHUNK, _CHUNK)], sem_o.at[k])
    for k in range(_NT - _NBUF, _NT):
        outs[k].wait()


_lookup = functools.partial(
    pl.kernel,
    out_type=(jax.ShapeDtypeStruct((_BATCH, _DIM), jnp.float32),
              jax.ShapeDtypeStruct((_BATCH, _DIM), jnp.float32)),
    mesh=plsc.VectorSubcoreMesh(core_axis_name="c", subcore_axis_name="s"),
    scratch_types=[
        pltpu.VMEM((_BPW,), jnp.int32),
        pltpu.VMEM((_BPW,), jnp.int32),
        pltpu.VMEM((_NBUF, _CHUNK, _DIM), jnp.float32),
        pltpu.VMEM_SHARED((_OFF_T + _ROWS, _DIM), jnp.float32),
        pltpu.SemaphoreType.DMA,
        pltpu.SemaphoreType.DMA((_NT,)),
        pltpu.SemaphoreType.DMA((_NT,)),
    ],
)(_tec_body)


def kernel(seconds_starts_totals, start_table, total_table):
    maxi = start_table.shape[0] - 1
    sst = jnp.clip(seconds_starts_totals, 0, maxi)
    out_s, out_t = _lookup(sst[:, 0], sst[:, 1] + _OFF_T,
                           start_table, total_table)
    return (out_s[:, None, :], out_t[:, None, :])


# speedup vs baseline: 1.0329x; 1.0329x over previous
import functools

import jax
import jax.numpy as jnp
from jax import lax
from jax.experimental import pallas as pl
from jax.experimental.pallas import tpu as pltpu
from jax.experimental.pallas import tpu_sc as plsc

_BATCH = 16384
_DIM = 128
_NC = 2
_NS = 16
_NW = _NC * _NS
_BPW = _BATCH // _NW
_CHUNK = 128
_NCHUNK = _BPW // _CHUNK
_NT = 2 * _NCHUNK
_NBUF = min(_NT - 1, 14)
_ROWS = 513
_OFF_T = 520


def _tec_body(starts_hbm, totals_hbm, start_hbm, total_hbm,
              out_s_hbm, out_t_hbm, idx_s_v, idx_t_v, rows_v, tabs_sh,
              sem_i, sem_g, sem_o):
    cid = lax.axis_index("c")
    sid = lax.axis_index("s")
    wid = sid * _NC + cid
    base = wid * _BPW
    idx_cps = [
        pltpu.async_copy(starts_hbm.at[pl.ds(base, _BPW)], idx_s_v, sem_i),
        pltpu.async_copy(totals_hbm.at[pl.ds(base, _BPW)], idx_t_v, sem_i),
    ]

    def _idx(k):
        ref = idx_s_v if k < _NCHUNK else idx_t_v
        return ref.at[pl.ds((k % _NCHUNK) * _CHUNK, _CHUNK)]

    @pl.when(sid == 0)
    def _():
        pltpu.sync_copy(start_hbm, tabs_sh.at[pl.ds(0, _ROWS)])

    @pl.when(sid == 1)
    def _():
        pltpu.sync_copy(total_hbm, tabs_sh.at[pl.ds(_OFF_T, _ROWS)])

    for cp in idx_cps:
        cp.wait()
    plsc.subcore_barrier()

    gathers = [None] * _NT
    outs = [None] * _NT
    for k in range(_NBUF):
        gathers[k] = pltpu.async_copy(
            tabs_sh.at[_idx(k)], rows_v.at[k], sem_g.at[k])
    for k in range(_NT):
        buf = k % _NBUF
        if k >= _NBUF:
            outs[k - _NBUF].wait()
            gathers[k] = pltpu.async_copy(
                tabs_sh.at[_idx(k)], rows_v.at[buf], sem_g.at[k])
        out = out_s_hbm if k < _NCHUNK else out_t_hbm
        gathers[k].wait()
        c = k % _NCHUNK
        outs[k] = pltpu.async_copy(
            rows_v.at[buf], out.at[pl.ds(base + c * _CHUNK, _CHUNK)], sem_o.at[k])
    for k in range(_NT - _NBUF, _NT):
        outs[k].wait()


_lookup = functools.partial(
    pl.kernel,
    out_type=(jax.ShapeDtypeStruct((_BATCH, _DIM), jnp.float32),
              jax.ShapeDtypeStruct((_BATCH, _DIM), jnp.float32)),
    mesh=plsc.VectorSubcoreMesh(core_axis_name="c", subcore_axis_name="s"),
    scratch_types=[
        pltpu.VMEM((_BPW,), jnp.int32),
        pltpu.VMEM((_BPW,), jnp.int32),
        pltpu.VMEM((_NBUF, _CHUNK, _DIM), jnp.float32),
        pltpu.VMEM_SHARED((_OFF_T + _ROWS, _DIM), jnp.float32),
        pltpu.SemaphoreType.DMA,
        pltpu.SemaphoreType.DMA((_NT,)),
        pltpu.SemaphoreType.DMA((_NT,)),
    ],
)(_tec_body)


def kernel(seconds_starts_totals, start_table, total_table):
    maxi = start_table.shape[0] - 1
    sst = jnp.clip(seconds_starts_totals, 0, maxi)
    out_s, out_t = _lookup(sst[:, 0], sst[:, 1] + _OFF_T,
                           start_table, total_table)
    return (out_s[:, None, :], out_t[:, None, :])
